# baseline (speedup 1.0000x reference)
import functools

import jax
import jax.numpy as jnp
from jax import lax
from jax.experimental import pallas as pl
from jax.experimental.pallas import tpu as pltpu
from jax.experimental.pallas import tpu_sc as plsc

N_ROWS = 16384
DEPTH = 1000
NC, NS, L = 2, 16, 16
NW = NC * NS
ROWS_PER_W = N_ROWS // NW
BLOCK_ROWS = 64
N_BLOCKS = ROWS_PER_W // BLOCK_ROWS
BLOCK_ELEMS = BLOCK_ROWS * DEPTH


def _onehot_body(x_hbm, out_hbm, idx_v, buf):
    wid = lax.axis_index("s") * NC + lax.axis_index("c")
    row0 = wid * ROWS_PER_W

    pltpu.sync_copy(x_hbm.at[pl.ds(row0 * 1, ROWS_PER_W)], idx_v)

    def zero_step(i, _):
        buf[pl.ds(i * L, L)] = jnp.zeros((L,), jnp.float32)
        return 0

    lax.fori_loop(0, BLOCK_ELEMS // L, zero_step, 0)

    ones = jnp.full((L,), 1.0, jnp.float32)
    zeros = jnp.zeros((L,), jnp.float32)
    lane = lax.iota(jnp.int32, L)

    for c in range(N_BLOCKS):
        flats = []
        for k in range(BLOCK_ROWS // L):
            cols = idx_v[pl.ds(c * BLOCK_ROWS + k * L, L)]
            flat = (k * L + lane) * DEPTH + cols
            flats.append(flat)
            plsc.store_scatter(buf, [flat], ones)
        pltpu.sync_copy(
            buf, out_hbm.at[pl.ds((row0 + c * BLOCK_ROWS) * DEPTH, BLOCK_ELEMS)]
        )
        for flat in flats:
            plsc.store_scatter(buf, [flat], zeros)


@jax.jit
def kernel(x):
    x = x.astype(jnp.int32)
    mesh = plsc.VectorSubcoreMesh(
        core_axis_name="c", subcore_axis_name="s", num_cores=NC, num_subcores=NS
    )
    out_flat = pl.kernel(
        _onehot_body,
        out_type=jax.ShapeDtypeStruct((N_ROWS * DEPTH,), jnp.float32),
        mesh=mesh,
        scratch_types=[
            pltpu.VMEM((ROWS_PER_W,), jnp.int32),
            pltpu.VMEM((BLOCK_ELEMS,), jnp.float32),
        ],
        compiler_params=pltpu.CompilerParams(needs_layout_passes=False),
    )(x)
    return out_flat.reshape(N_ROWS, DEPTH)


# speedup vs baseline: 1.0025x; 1.0025x over previous
import jax
import jax.numpy as jnp
from jax import lax
from jax.experimental import pallas as pl
from jax.experimental.pallas import tpu as pltpu
from jax.experimental.pallas import tpu_sc as plsc

N_ROWS = 16384
DEPTH = 1000
NC, NS, L = 2, 16, 16
NW = NC * NS
ROWS_PER_W = N_ROWS // NW
TMPL_ROWS = 64
TMPL_ELEMS = TMPL_ROWS * DEPTH
N_BLOCKS = ROWS_PER_W // TMPL_ROWS
SCAT = 128
N_SCAT = ROWS_PER_W // SCAT


def _onehot_body(x_hbm, out_hbm, idx_v, zbuf, ones_v, flat_v, sem_z, sem_s):
    wid = lax.axis_index("s") * NC + lax.axis_index("c")
    row0 = wid * ROWS_PER_W

    pltpu.sync_copy(x_hbm.at[pl.ds(row0, ROWS_PER_W)], idx_v)

    zeros = jnp.zeros((L,), jnp.float32)

    def zero_step(i, _):
        for u in range(8):
            zbuf[pl.ds(i * 8 * L + u * L, L)] = zeros
        return 0

    lax.fori_loop(0, TMPL_ELEMS // (8 * L), zero_step, 0)

    ones = jnp.full((L,), 1.0, jnp.float32)
    for u in range(SCAT // L):
        ones_v[pl.ds(u * L, L)] = ones

    lane = lax.iota(jnp.int32, L)
    for j in range(N_SCAT):
        for k in range(SCAT // L):
            g = j * SCAT + k * L
            cols = idx_v[pl.ds(g, L)]
            flat_v[j, pl.ds(k * L, L)] = (row0 + g + lane) * DEPTH + cols

    copies = [
        pltpu.make_async_copy(
            zbuf,
            out_hbm.at[pl.ds((row0 + c * TMPL_ROWS) * DEPTH, TMPL_ELEMS)],
            sem_z,
        )
        for c in range(N_BLOCKS)
    ]
    for cp in copies:
        cp.start()
    for cp in copies:
        cp.wait()

    scats = [
        pltpu.make_async_copy(ones_v, out_hbm.at[flat_v.at[j]], sem_s)
        for j in range(N_SCAT)
    ]
    for cp in scats:
        cp.start()
    for cp in scats:
        cp.wait()


@jax.jit
def kernel(x):
    x = x.astype(jnp.int32)
    mesh = plsc.VectorSubcoreMesh(
        core_axis_name="c", subcore_axis_name="s", num_cores=NC, num_subcores=NS
    )
    out_flat = pl.kernel(
        _onehot_body,
        out_type=jax.ShapeDtypeStruct((N_ROWS * DEPTH,), jnp.float32),
        mesh=mesh,
        scratch_types=[
            pltpu.VMEM((ROWS_PER_W,), jnp.int32),
            pltpu.VMEM((TMPL_ELEMS,), jnp.float32),
            pltpu.VMEM((SCAT,), jnp.float32),
            pltpu.VMEM((N_SCAT, SCAT), jnp.int32),
            pltpu.SemaphoreType.DMA,
            pltpu.SemaphoreType.DMA,
        ],
        compiler_params=pltpu.CompilerParams(needs_layout_passes=False),
    )(x)
    return out_flat.reshape(N_ROWS, DEPTH)


# speedup vs baseline: 1.7895x; 1.7850x over previous
import jax
import jax.numpy as jnp
from jax import lax
from jax.experimental import pallas as pl
from jax.experimental.pallas import tpu as pltpu
from jax.experimental.pallas import tpu_sc as plsc

N_ROWS = 16384
DEPTH = 1000
NC, NS, L = 2, 16, 16
NW = NC * NS
ROWS_PER_W = N_ROWS // NW
BLK = 32
N_BLOCKS = ROWS_PER_W // BLK
NBUF = 2


def _onehot_body(x_hbm, out_hbm, idx_v, buf0, buf1, sem0, sem1):
    bufs = [buf0, buf1]
    sems = [sem0, sem1]
    wid = lax.axis_index("s") * NC + lax.axis_index("c")
    row0 = wid * ROWS_PER_W

    pltpu.sync_copy(x_hbm.at[pl.ds(row0, ROWS_PER_W)], idx_v)

    zeros = jnp.zeros((L,), jnp.float32)

    def zero_step(r, _):
        for u in range(DEPTH // L):
            buf0[r, pl.ds(u * L, L)] = zeros
            buf1[r, pl.ds(u * L, L)] = zeros
        buf0[r, pl.ds(DEPTH - L, L)] = zeros
        buf1[r, pl.ds(DEPTH - L, L)] = zeros
        return 0

    lax.fori_loop(0, BLK, zero_step, 0)

    ones = jnp.full((L,), 1.0, jnp.float32)
    lane = lax.iota(jnp.int32, L)

    def idxs_for(c):
        out = []
        for k in range(BLK // L):
            cols = idx_v[pl.ds(c * BLK + k * L, L)]
            out.append((k * L + lane, cols))
        return out

    copies = [None] * N_BLOCKS
    idxs = [None] * N_BLOCKS
    for c in range(N_BLOCKS):
        b = c % NBUF
        if c >= NBUF:
            copies[c - NBUF].wait()
            for rows, cols in idxs[c - NBUF]:
                plsc.store_scatter(bufs[b], [rows, cols], zeros)
        idxs[c] = idxs_for(c)
        for rows, cols in idxs[c]:
            plsc.store_scatter(bufs[b], [rows, cols], ones)
        cp = pltpu.make_async_copy(
            bufs[b], out_hbm.at[pl.ds(row0 + c * BLK, BLK)], sems[b]
        )
        cp.start()
        copies[c] = cp
    for c in range(N_BLOCKS - NBUF, N_BLOCKS):
        copies[c].wait()


@jax.jit
def kernel(x):
    x = x.astype(jnp.int32)
    mesh = plsc.VectorSubcoreMesh(
        core_axis_name="c", subcore_axis_name="s", num_cores=NC, num_subcores=NS
    )
    return pl.kernel(
        _onehot_body,
        out_type=jax.ShapeDtypeStruct((N_ROWS, DEPTH), jnp.float32),
        mesh=mesh,
        scratch_types=[
            pltpu.VMEM((ROWS_PER_W,), jnp.int32),
            pltpu.VMEM((BLK, DEPTH), jnp.float32),
            pltpu.VMEM((BLK, DEPTH), jnp.float32),
            pltpu.SemaphoreType.DMA,
            pltpu.SemaphoreType.DMA,
        ],
        compiler_params=pltpu.CompilerParams(needs_layout_passes=False),
    )(x)


# speedup vs baseline: 4.0562x; 2.2666x over previous
import jax
import jax.numpy as jnp
from jax import lax
from jax.experimental import pallas as pl
from jax.experimental.pallas import tpu as pltpu
from jax.experimental.pallas import tpu_sc as plsc

N_ROWS = 16384
DEPTH = 1000
NC, NS, L = 2, 16, 16
NW = NC * NS
COLS_PER_W = N_ROWS // NW
KBLK = 40
N_BLOCKS = DEPTH // KBLK
NBUF = 2


def _onehot_body(x_hbm, out_hbm, idx_v, buf0, buf1, sem0, sem1):
    bufs = [buf0, buf1]
    sems = [sem0, sem1]
    wid = lax.axis_index("s") * NC + lax.axis_index("c")
    col0 = wid * COLS_PER_W

    pltpu.sync_copy(x_hbm.at[pl.ds(col0, COLS_PER_W)], idx_v)

    zeros = jnp.zeros((L,), jnp.float32)

    def zero_step(r, _):
        for u in range(COLS_PER_W // L):
            buf0[r, pl.ds(u * L, L)] = zeros
            buf1[r, pl.ds(u * L, L)] = zeros
        return 0

    lax.fori_loop(0, KBLK, zero_step, 0)

    ones = jnp.full((L,), 1.0, jnp.float32)
    lane = lax.iota(jnp.int32, L)

    def scatter_pass(buf, c, val):
        j0 = c * KBLK

        def step(g, _):
            xv = idx_v[pl.ds(g * L, L)]
            rows = xv - j0
            m = (xv >= j0) & (xv < j0 + KBLK)
            cols = g * L + lane
            plsc.store_scatter(buf, [rows, cols], val, mask=m)
            return 0

        lax.fori_loop(0, COLS_PER_W // L, step, 0)

    copies = [None] * N_BLOCKS
    for c in range(N_BLOCKS):
        b = c % NBUF
        if c >= NBUF:
            copies[c - NBUF].wait()
            scatter_pass(bufs[b], c - NBUF, zeros)
        scatter_pass(bufs[b], c, ones)
        cp = pltpu.make_async_copy(
            bufs[b],
            out_hbm.at[pl.ds(c * KBLK, KBLK), pl.ds(col0, COLS_PER_W)],
            sems[b],
        )
        cp.start()
        copies[c] = cp
    for c in range(N_BLOCKS - NBUF, N_BLOCKS):
        copies[c].wait()


@jax.jit
def kernel(x):
    x = x.astype(jnp.int32)
    mesh = plsc.VectorSubcoreMesh(
        core_axis_name="c", subcore_axis_name="s", num_cores=NC, num_subcores=NS
    )
    out_t = pl.kernel(
        _onehot_body,
        out_type=jax.ShapeDtypeStruct((DEPTH, N_ROWS), jnp.float32),
        mesh=mesh,
        scratch_types=[
            pltpu.VMEM((COLS_PER_W,), jnp.int32),
            pltpu.VMEM((KBLK, COLS_PER_W), jnp.float32),
            pltpu.VMEM((KBLK, COLS_PER_W), jnp.float32),
            pltpu.SemaphoreType.DMA,
            pltpu.SemaphoreType.DMA,
        ],
        compiler_params=pltpu.CompilerParams(needs_layout_passes=False),
    )(x)
    return out_t.T


# speedup vs baseline: 4.1371x; 1.0200x over previous
import jax
import jax.numpy as jnp
from jax import lax
from jax.experimental import pallas as pl
from jax.experimental.pallas import tpu as pltpu
from jax.experimental.pallas import tpu_sc as plsc

N_ROWS = 16384
DEPTH = 1000
NC, NS, L = 2, 16, 16
NW = NC * NS
COLS_PER_W = N_ROWS // NW
KBLK = 40
N_BLOCKS = DEPTH // KBLK
NBUF = 4


def _onehot_body(x_hbm, out_hbm, idx_v, buf0, buf1, buf2, buf3,
                 sem0, sem1, sem2, sem3):
    bufs = [buf0, buf1, buf2, buf3]
    sems = [sem0, sem1, sem2, sem3]
    wid = lax.axis_index("s") * NC + lax.axis_index("c")
    col0 = wid * COLS_PER_W

    pltpu.sync_copy(x_hbm.at[pl.ds(col0, COLS_PER_W)], idx_v)

    zeros = jnp.zeros((L,), jnp.float32)
    ones = jnp.full((L,), 1.0, jnp.float32)
    lane = lax.iota(jnp.int32, L)

    def zero_fill(buf):
        def zstep(r, _):
            for u in range(COLS_PER_W // L):
                buf[r, pl.ds(u * L, L)] = zeros
            return 0

        lax.fori_loop(0, KBLK, zstep, 0)

    def set_pass(buf, c):
        j0 = c * KBLK

        def step(g, _):
            xv = idx_v[pl.ds(g * L, L)]
            cols = g * L + lane
            m = (xv >= j0) & (xv < j0 + KBLK)
            plsc.store_scatter(buf, [xv - j0, cols], ones, mask=m)
            return 0

        lax.fori_loop(0, COLS_PER_W // L, step, 0)

    def clear_set_pass(buf, c_old, c_new):
        j0o = c_old * KBLK
        j0n = c_new * KBLK

        def step(g, _):
            xv = idx_v[pl.ds(g * L, L)]
            cols = g * L + lane
            mo = (xv >= j0o) & (xv < j0o + KBLK)
            plsc.store_scatter(buf, [xv - j0o, cols], zeros, mask=mo)
            mn = (xv >= j0n) & (xv < j0n + KBLK)
            plsc.store_scatter(buf, [xv - j0n, cols], ones, mask=mn)
            return 0

        lax.fori_loop(0, COLS_PER_W // L, step, 0)

    def start_dma(b, c):
        cp = pltpu.make_async_copy(
            bufs[b],
            out_hbm.at[pl.ds(c * KBLK, KBLK), pl.ds(col0, COLS_PER_W)],
            sems[b],
        )
        cp.start()
        return cp

    copies = [None] * N_BLOCKS
    for c in range(NBUF):
        zero_fill(bufs[c])
        set_pass(bufs[c], c)
        copies[c] = start_dma(c, c)
    for c in range(NBUF, N_BLOCKS):
        b = c % NBUF
        copies[c - NBUF].wait()
        clear_set_pass(bufs[b], c - NBUF, c)
        copies[c] = start_dma(b, c)
    for c in range(N_BLOCKS - NBUF, N_BLOCKS):
        copies[c].wait()


@jax.jit
def kernel(x):
    x = x.astype(jnp.int32)
    mesh = plsc.VectorSubcoreMesh(
        core_axis_name="c", subcore_axis_name="s", num_cores=NC, num_subcores=NS
    )
    out_t = pl.kernel(
        _onehot_body,
        out_type=jax.ShapeDtypeStruct((DEPTH, N_ROWS), jnp.float32),
        mesh=mesh,
        scratch_types=[
            pltpu.VMEM((COLS_PER_W,), jnp.int32),
            pltpu.VMEM((KBLK, COLS_PER_W), jnp.float32),
            pltpu.VMEM((KBLK, COLS_PER_W), jnp.float32),
            pltpu.VMEM((KBLK, COLS_PER_W), jnp.float32),
            pltpu.VMEM((KBLK, COLS_PER_W), jnp.float32),
            pltpu.SemaphoreType.DMA,
            pltpu.SemaphoreType.DMA,
            pltpu.SemaphoreType.DMA,
            pltpu.SemaphoreType.DMA,
        ],
        compiler_params=pltpu.CompilerParams(needs_layout_passes=False),
    )(x)
    return out_t.T
